# baseline (speedup 1.0000x reference)
import jax
import jax.numpy as jnp
from jax import lax
from jax.experimental import pallas as pl
from jax.experimental.pallas import tpu as pltpu
from jax.experimental.pallas import tpu_sc as plsc

_NC = 2
_NS = 16
_NW = _NC * _NS

_CH = 128
_NB = 5
_K = 3


def _gather_kernel(table, idxh, out, idx_v, bufs, gsem, wsem):
    n_chunks = idxh.shape[1]
    b_per_w = n_chunks * _CH
    n_groups = n_chunks // _NB
    wid = lax.axis_index("s") * _NC + lax.axis_index("c")
    base = wid * b_per_w
    pltpu.sync_copy(idxh.at[wid], idx_v)

    def idx_slice(c):
        return idx_v.at[c]

    def fire_gather(c, b):
        pltpu.async_copy(table.at[idx_slice(c)], bufs.at[b], gsem.at[b])

    def wait_gather(c, b):
        pltpu.make_async_copy(table.at[idx_slice(c)], bufs.at[b],
                              gsem.at[b]).wait()

    def fire_wb(c, b):
        pltpu.async_copy(bufs.at[b], out.at[pl.ds(base + c * _CH, _CH)],
                         wsem.at[b])

    def wait_wb(c, b):
        pltpu.make_async_copy(bufs.at[b], out.at[pl.ds(base + c * _CH, _CH)],
                              wsem.at[b]).wait()

    for c in range(_K):
        fire_gather(c, c % _NB)

    for b in range(_NB):
        i = b
        bb = (b + _K) % _NB
        if i + _K >= _NB:
            wait_wb(i + _K - _NB, bb)
        fire_gather(i + _K, bb)
        wait_gather(i, b)
        fire_wb(i, b)

    def group_body(g, carry):
        for b in range(_NB):
            i = g * _NB + b
            bb = (b + _K) % _NB
            wait_wb(i + _K - _NB, bb)
            fire_gather(i + _K, bb)
            wait_gather(i, b)
            fire_wb(i, b)
        return carry

    lax.fori_loop(1, n_groups - 1, group_body, 0, unroll=False)

    for b in range(_NB):
        i = (n_groups - 1) * _NB + b
        if i + _K < n_chunks:
            bb = (b + _K) % _NB
            wait_wb(i + _K - _NB, bb)
            fire_gather(i + _K, bb)
        wait_gather(i, b)
        fire_wb(i, b)

    for b in range(_NB):
        wait_wb(n_chunks - _NB + b, b)


@jax.jit
def _embedding_lookup(weight, flat_ids):
    b_total = flat_ids.shape[0]
    d = weight.shape[1]
    b_per_w = b_total // _NW
    mesh = plsc.VectorSubcoreMesh(core_axis_name="c", subcore_axis_name="s")
    f = pl.kernel(
        _gather_kernel,
        out_type=jax.ShapeDtypeStruct((b_total, d), jnp.float32),
        mesh=mesh,
        scratch_types=[
            pltpu.VMEM((b_per_w // _CH, _CH), jnp.int32),
            pltpu.VMEM((_NB, _CH, d), jnp.float32),
            pltpu.SemaphoreType.DMA((_NB,)),
            pltpu.SemaphoreType.DMA((_NB,)),
        ],
    )
    return f(weight, flat_ids.reshape(_NW, b_per_w // _CH, _CH))


def kernel(token_ids, weight):
    b, l = token_ids.shape
    flat = token_ids.reshape(-1).astype(jnp.int32)
    out = _embedding_lookup(weight, flat)
    return out.reshape(b, l, weight.shape[1])


# speedup vs baseline: 1.0009x; 1.0009x over previous
import jax
import jax.numpy as jnp
from jax import lax
from jax.experimental import pallas as pl
from jax.experimental.pallas import tpu as pltpu
from jax.experimental.pallas import tpu_sc as plsc

_NC = 2
_NS = 16
_NW = _NC * _NS

_CH = 128
_NB = 5
_K = 4


def _gather_kernel(table, idxh, out, idx_v, bufs, gsem, wsem):
    n_chunks = idxh.shape[1]
    b_per_w = n_chunks * _CH
    n_groups = n_chunks // _NB
    wid = lax.axis_index("s") * _NC + lax.axis_index("c")
    base = wid * b_per_w
    pltpu.sync_copy(idxh.at[wid], idx_v)

    def idx_slice(c):
        return idx_v.at[c]

    def fire_gather(c, b):
        pltpu.async_copy(table.at[idx_slice(c)], bufs.at[b], gsem.at[b])

    def wait_gather(c, b):
        pltpu.make_async_copy(table.at[idx_slice(c)], bufs.at[b],
                              gsem.at[b]).wait()

    def fire_wb(c, b):
        pltpu.async_copy(bufs.at[b], out.at[pl.ds(base + c * _CH, _CH)],
                         wsem.at[b])

    def wait_wb(c, b):
        pltpu.make_async_copy(bufs.at[b], out.at[pl.ds(base + c * _CH, _CH)],
                              wsem.at[b]).wait()

    for c in range(_K):
        fire_gather(c, c % _NB)

    for b in range(_NB):
        i = b
        bb = (b + _K) % _NB
        if i + _K >= _NB:
            wait_wb(i + _K - _NB, bb)
        fire_gather(i + _K, bb)
        wait_gather(i, b)
        fire_wb(i, b)

    def group_body(g, carry):
        for b in range(_NB):
            i = g * _NB + b
            bb = (b + _K) % _NB
            wait_wb(i + _K - _NB, bb)
            fire_gather(i + _K, bb)
            wait_gather(i, b)
            fire_wb(i, b)
        return carry

    lax.fori_loop(1, n_groups - 1, group_body, 0, unroll=False)

    for b in range(_NB):
        i = (n_groups - 1) * _NB + b
        if i + _K < n_chunks:
            bb = (b + _K) % _NB
            wait_wb(i + _K - _NB, bb)
            fire_gather(i + _K, bb)
        wait_gather(i, b)
        fire_wb(i, b)

    for b in range(_NB):
        wait_wb(n_chunks - _NB + b, b)


@jax.jit
def _embedding_lookup(weight, flat_ids):
    b_total = flat_ids.shape[0]
    d = weight.shape[1]
    b_per_w = b_total // _NW
    mesh = plsc.VectorSubcoreMesh(core_axis_name="c", subcore_axis_name="s")
    f = pl.kernel(
        _gather_kernel,
        out_type=jax.ShapeDtypeStruct((b_total, d), jnp.float32),
        mesh=mesh,
        scratch_types=[
            pltpu.VMEM((b_per_w // _CH, _CH), jnp.int32),
            pltpu.VMEM((_NB, _CH, d), jnp.float32),
            pltpu.SemaphoreType.DMA((_NB,)),
            pltpu.SemaphoreType.DMA((_NB,)),
        ],
    )
    return f(weight, flat_ids.reshape(_NW, b_per_w // _CH, _CH))


def kernel(token_ids, weight):
    b, l = token_ids.shape
    flat = token_ids.reshape(-1).astype(jnp.int32)
    out = _embedding_lookup(weight, flat)
    return out.reshape(b, l, weight.shape[1])


# speedup vs baseline: 1.0082x; 1.0072x over previous
import jax
import jax.numpy as jnp
from jax import lax
from jax.experimental import pallas as pl
from jax.experimental.pallas import tpu as pltpu
from jax.experimental.pallas import tpu_sc as plsc

_NC = 2
_NS = 16
_NW = _NC * _NS

_G = 2
_NB = 3
_K = 2


def _gather_kernel(table, idxh, out, idx_v, bufs, gsem, wsem):
    n_rows = idxh.shape[1]
    n_chunks = n_rows // _G
    wid = lax.axis_index("s") * _NC + lax.axis_index("c")
    base = wid * n_rows
    pltpu.sync_copy(idxh.at[wid], idx_v)

    def fire_gather(c, b):
        for g in range(_G):
            pltpu.async_copy(table.at[idx_v.at[c * _G + g]], bufs.at[b, g],
                             gsem.at[b, g])

    def wait_gather(c, b):
        for g in range(_G):
            pltpu.make_async_copy(table.at[idx_v.at[c * _G + g]],
                                  bufs.at[b, g], gsem.at[b, g]).wait()

    def fire_wb(c, b):
        pltpu.async_copy(bufs.at[b], out.at[pl.ds(base + c * _G, _G)],
                         wsem.at[b])

    def wait_wb(c, b):
        pltpu.make_async_copy(bufs.at[b], out.at[pl.ds(base + c * _G, _G)],
                              wsem.at[b]).wait()

    for c in range(_K):
        fire_gather(c, c)

    def chunk_body(c, carry):
        @pl.when(c + _K < n_chunks)
        def _():
            bb = lax.rem(c + _K, _NB)

            @pl.when(c + _K >= _NB)
            def _():
                wait_wb(c + _K - _NB, bb)

            fire_gather(c + _K, bb)

        b = lax.rem(c, _NB)
        wait_gather(c, b)
        fire_wb(c, b)
        return carry

    lax.fori_loop(0, n_chunks, chunk_body, 0, unroll=False)

    for r in range(_NB):
        c = n_chunks - _NB + r
        wait_wb(c, lax.rem(c, _NB))


@jax.jit
def _embedding_lookup(weight, flat_ids):
    b_total = flat_ids.shape[0]
    d = weight.shape[1]
    rows_per_w = b_total // _NW // 128
    mesh = plsc.VectorSubcoreMesh(core_axis_name="c", subcore_axis_name="s")
    f = pl.kernel(
        _gather_kernel,
        out_type=jax.ShapeDtypeStruct((b_total // 128, 128, d), jnp.float32),
        mesh=mesh,
        scratch_types=[
            pltpu.VMEM((rows_per_w, 128), jnp.int32),
            pltpu.VMEM((_NB, _G, 128, d), jnp.float32),
            pltpu.SemaphoreType.DMA((_NB, _G)),
            pltpu.SemaphoreType.DMA((_NB,)),
        ],
    )
    return f(weight, flat_ids.reshape(_NW, rows_per_w, 128))


def kernel(token_ids, weight):
    b, l = token_ids.shape
    flat = token_ids.reshape(-1).astype(jnp.int32)
    out = _embedding_lookup(weight, flat)
    return out.reshape(b, l, weight.shape[1])


# speedup vs baseline: 1.0135x; 1.0053x over previous
import jax
import jax.numpy as jnp
from jax import lax
from jax.experimental import pallas as pl
from jax.experimental.pallas import tpu as pltpu
from jax.experimental.pallas import tpu_sc as plsc

_NC = 2
_NS = 16
_NW = _NC * _NS

_G = 1
_NB = 7
_K = 6


def _gather_kernel(table, idxh, out, idx_v, bufs, gsem, wsem):
    n_rows = idxh.shape[1]
    n_chunks = n_rows // _G
    wid = lax.axis_index("s") * _NC + lax.axis_index("c")
    base = wid * n_rows
    pltpu.sync_copy(idxh.at[wid], idx_v)

    def fire_gather(c, b):
        for g in range(_G):
            pltpu.async_copy(table.at[idx_v.at[c * _G + g]], bufs.at[b, g],
                             gsem.at[b, g])

    def wait_gather(c, b):
        for g in range(_G):
            pltpu.make_async_copy(table.at[idx_v.at[c * _G + g]],
                                  bufs.at[b, g], gsem.at[b, g]).wait()

    def fire_wb(c, b):
        pltpu.async_copy(bufs.at[b], out.at[pl.ds(base + c * _G, _G)],
                         wsem.at[b])

    def wait_wb(c, b):
        pltpu.make_async_copy(bufs.at[b], out.at[pl.ds(base + c * _G, _G)],
                              wsem.at[b]).wait()

    for c in range(_K):
        fire_gather(c, c)

    def chunk_body(c, carry):
        @pl.when(c + _K < n_chunks)
        def _():
            bb = lax.rem(c + _K, _NB)

            @pl.when(c + _K >= _NB)
            def _():
                wait_wb(c + _K - _NB, bb)

            fire_gather(c + _K, bb)

        b = lax.rem(c, _NB)
        wait_gather(c, b)
        fire_wb(c, b)
        return carry

    lax.fori_loop(0, n_chunks, chunk_body, 0, unroll=False)

    for r in range(_NB):
        c = n_chunks - _NB + r
        wait_wb(c, lax.rem(c, _NB))


@jax.jit
def _embedding_lookup(weight, flat_ids):
    b_total = flat_ids.shape[0]
    d = weight.shape[1]
    rows_per_w = b_total // _NW // 128
    mesh = plsc.VectorSubcoreMesh(core_axis_name="c", subcore_axis_name="s")
    f = pl.kernel(
        _gather_kernel,
        out_type=jax.ShapeDtypeStruct((b_total // 128, 128, d), jnp.float32),
        mesh=mesh,
        scratch_types=[
            pltpu.VMEM((rows_per_w, 128), jnp.int32),
            pltpu.VMEM((_NB, _G, 128, d), jnp.float32),
            pltpu.SemaphoreType.DMA((_NB, _G)),
            pltpu.SemaphoreType.DMA((_NB,)),
        ],
    )
    return f(weight, flat_ids.reshape(_NW, rows_per_w, 128))


def kernel(token_ids, weight):
    b, l = token_ids.shape
    flat = token_ids.reshape(-1).astype(jnp.int32)
    out = _embedding_lookup(weight, flat)
    return out.reshape(b, l, weight.shape[1])
